# baseline (speedup 1.0000x reference)
import functools

import jax
import jax.numpy as jnp
from jax import lax
from jax.experimental import pallas as pl
from jax.experimental.pallas import tpu as pltpu
from jax.experimental.pallas import tpu_sc as plsc

_N = 8192
_BM = 256
_K = 9
_IDX_MASK = -8192

_SC_ROWS = 5632
_TC_ALIGN_ROWS = _N - _SC_ROWS
_NW = 32
_RPW = _SC_ROWS // _NW
_CHUNKS = _N // 16


def _tc_body(p_ref, pT_ref, r_ref, rT_ref, s_ref, o_ref, out_ref, acc_ref):
    i = pl.program_id(0)
    nb = pl.num_programs(0)

    @pl.when(i == 0)
    def _init():
        acc_ref[0] = 0.0
        acc_ref[1] = 0.0
        acc_ref[2] = 0.0
        acc_ref[3] = 0.0

    p_blk = p_ref[...]
    pT = pT_ref[...]
    cross = jnp.dot(p_blk, pT, preferred_element_type=jnp.float32)
    sq_i = jnp.sum(p_blk * p_blk, axis=1, keepdims=True)
    sq_j = jnp.sum(pT * pT, axis=0, keepdims=True)
    d2 = jnp.maximum(sq_i + (sq_j - 2.0 * cross), 1e-12)

    @pl.when(i < _TC_ALIGN_ROWS // _BM)
    def _align():
        rd = jnp.dot(r_ref[...], rT_ref[...], preferred_element_type=jnp.float32)
        nearby = (d2 < 4e-4) & (d2 > 1e-10)
        acc_ref[0] += jnp.sum(jnp.where(nearby, 1.0 - jnp.abs(rd), 0.0))
        acc_ref[1] += jnp.sum(nearby.astype(jnp.float32))

    bits = jax.lax.bitcast_convert_type(d2, jnp.int32)
    iota = jax.lax.broadcasted_iota(jnp.int32, d2.shape, 1)
    w = jax.lax.bitcast_convert_type(
        jnp.bitwise_or(jnp.bitwise_and(bits, _IDX_MASK), iota), jnp.float32)

    def _val(m):
        d2t = jax.lax.bitcast_convert_type(
            jnp.bitwise_and(jax.lax.bitcast_convert_type(m, jnp.int32), _IDX_MASK),
            jnp.float32)
        return jnp.sqrt(d2t)

    m = jnp.min(w, axis=1, keepdims=True)
    dmin = _val(m)
    sum9 = dmin
    for _ in range(_K - 1):
        m = jnp.min(jnp.where(w > m, w, 1e30), axis=1, keepdims=True)
        sum9 = sum9 + _val(m)
    avg = (sum9 - dmin) * 0.125
    acc_ref[2] += jnp.sum(jnp.abs(avg - 0.01) * o_ref[:, 0:1])

    a = jnp.exp(s_ref[...])
    a0 = a[:, 0:1]
    a1 = a[:, 1:2]
    a2 = a[:, 2:3]
    lo = jnp.minimum(jnp.minimum(a0, a1), a2)
    hi = jnp.maximum(jnp.maximum(a0, a1), a2)
    mid = jnp.maximum(jnp.minimum(a0, a1), jnp.minimum(jnp.maximum(a0, a1), a2))
    fr = jnp.log(hi / (lo + 1e-8) + 1e-8)
    disc = 1.0 / (jnp.abs(hi - mid) + 0.001)
    acc_ref[3] += jnp.sum(fr + 0.1 * disc)

    @pl.when(i == nb - 1)
    def _finish():
        out_ref[...] = jnp.array([[acc_ref[0], acc_ref[1], acc_ref[2], acc_ref[3]]],
                                 jnp.float32)


_sc_mesh = plsc.VectorSubcoreMesh(core_axis_name="c", subcore_axis_name="s")


@functools.partial(
    pl.kernel,
    mesh=_sc_mesh,
    out_type=jax.ShapeDtypeStruct((_NW, 2, 16), jnp.float32),
    scratch_types=[
        pltpu.VMEM((7, _N), jnp.float32),
        pltpu.VMEM((16,), jnp.float32),
        pltpu.VMEM((16,), jnp.float32),
    ],
)
def _sc_align(cols_hbm, out_hbm, cols_v, asum_v, cnt_v):
    wid = lax.axis_index("s") * 2 + lax.axis_index("c")
    pltpu.sync_copy(cols_hbm, cols_v)
    base = _N - _SC_ROWS + wid * _RPW

    def group_body(g, group_carry):
        start = base + g * 16
        rvec = [cols_v[k, pl.ds(start, 16)] for k in range(7)]
        acc_g, cnt_g = group_carry
        for r in range(0, 16, 2):
            sca = [rv[r] for rv in rvec]
            scb = [rv[r + 1] for rv in rvec]

            def chunk_body(j, carry):
                a, c = carry
                s = j * 16
                vx = cols_v[0, pl.ds(s, 16)]
                vy = cols_v[1, pl.ds(s, 16)]
                vz = cols_v[2, pl.ds(s, 16)]
                v0 = cols_v[3, pl.ds(s, 16)]
                v1 = cols_v[4, pl.ds(s, 16)]
                v2 = cols_v[5, pl.ds(s, 16)]
                v3 = cols_v[6, pl.ds(s, 16)]
                for sc in (sca, scb):
                    dx = vx - sc[0]
                    dy = vy - sc[1]
                    dz = vz - sc[2]
                    d2 = dx * dx + dy * dy + dz * dz
                    qd = v0 * sc[3] + v1 * sc[4] + v2 * sc[5] + v3 * sc[6]
                    mask = (d2 < 4e-4) & (d2 > 1e-10)
                    a = a + jnp.where(mask, 1.0 - jnp.abs(qd), 0.0)
                    c = c + jnp.where(mask, 1.0, 0.0)
                return a, c

            acc_g, cnt_g = lax.fori_loop(0, _CHUNKS, chunk_body,
                                         (acc_g, cnt_g), unroll=4)
        return acc_g, cnt_g

    zeros = jnp.zeros((16,), jnp.float32)
    acc, cnt = lax.fori_loop(0, _RPW // 16, group_body, (zeros, zeros))
    asum_v[...] = acc
    cnt_v[...] = cnt
    pltpu.sync_copy(asum_v, out_hbm.at[wid, 0])
    pltpu.sync_copy(cnt_v, out_hbm.at[wid, 1])


def kernel(positions, rotations, scales, opacity):
    pT = positions.T
    rT = rotations.T
    cols = jnp.concatenate([pT, rT], axis=0)

    sc_part = _sc_align(cols)

    tc_part = pl.pallas_call(
        _tc_body,
        grid=(_N // _BM,),
        in_specs=[
            pl.BlockSpec((_BM, 3), lambda i: (i, 0)),
            pl.BlockSpec((3, _N), lambda i: (0, 0)),
            pl.BlockSpec((_BM, 4), lambda i: (i, 0)),
            pl.BlockSpec((4, _N), lambda i: (0, 0)),
            pl.BlockSpec((_BM, 3), lambda i: (i, 0)),
            pl.BlockSpec((_BM, 1), lambda i: (i, 0)),
        ],
        out_specs=pl.BlockSpec((1, 4), lambda i: (0, 0)),
        out_shape=jax.ShapeDtypeStruct((1, 4), jnp.float32),
        scratch_shapes=[pltpu.SMEM((4,), jnp.float32)],
    )(positions, pT, rotations, rT, scales, opacity)

    asum = tc_part[0, 0] + jnp.sum(sc_part[:, 0, :])
    cnt = tc_part[0, 1] + jnp.sum(sc_part[:, 1, :])
    flat = -tc_part[0, 3] / _N
    align = jnp.where(cnt > 0.0, asum / jnp.maximum(cnt, 1.0), 0.0)
    dens = tc_part[0, 2] / _N
    return flat + 0.5 * align + 0.2 * dens


# speedup vs baseline: 1.1720x; 1.1720x over previous
import functools

import jax
import jax.numpy as jnp
from jax import lax
from jax.experimental import pallas as pl
from jax.experimental.pallas import tpu as pltpu
from jax.experimental.pallas import tpu_sc as plsc

_N = 8192
_BM = 256
_K = 9
_IDX_MASK = -8192

_SC_ROWS = 4608
_TC_ALIGN_ROWS = _N - _SC_ROWS
_NW = 32
_RPW = _SC_ROWS // _NW
_CHUNKS = _N // 16


def _tc_body(p_ref, pT_ref, r_ref, rT_ref, s_ref, o_ref, out_ref, acc_ref):
    i = pl.program_id(0)
    nb = pl.num_programs(0)

    @pl.when(i == 0)
    def _init():
        acc_ref[0] = 0.0
        acc_ref[1] = 0.0
        acc_ref[2] = 0.0
        acc_ref[3] = 0.0

    p_blk = p_ref[...]
    pT = pT_ref[...]
    cross = jnp.dot(p_blk, pT, preferred_element_type=jnp.float32)
    sq_i = jnp.sum(p_blk * p_blk, axis=1, keepdims=True)
    sq_j = jnp.sum(pT * pT, axis=0, keepdims=True)
    d2 = jnp.maximum(sq_i + (sq_j - 2.0 * cross), 1e-12)

    @pl.when(i < _TC_ALIGN_ROWS // _BM)
    def _align():
        rd = jnp.dot(r_ref[...], rT_ref[...], preferred_element_type=jnp.float32)
        nearby = (d2 < 4e-4) & (d2 > 1e-10)
        acc_ref[0] += jnp.sum(jnp.where(nearby, 1.0 - jnp.abs(rd), 0.0))
        acc_ref[1] += jnp.sum(nearby.astype(jnp.float32))

    bits = jax.lax.bitcast_convert_type(d2, jnp.int32)
    iota = jax.lax.broadcasted_iota(jnp.int32, d2.shape, 1)
    w = jax.lax.bitcast_convert_type(
        jnp.bitwise_or(jnp.bitwise_and(bits, _IDX_MASK), iota), jnp.float32)

    def _val(m):
        d2t = jax.lax.bitcast_convert_type(
            jnp.bitwise_and(jax.lax.bitcast_convert_type(m, jnp.int32), _IDX_MASK),
            jnp.float32)
        return jnp.sqrt(d2t)

    m = jnp.min(w, axis=1, keepdims=True)
    dmin = _val(m)
    sum9 = dmin
    for _ in range(_K - 1):
        m = jnp.min(jnp.where(w > m, w, 1e30), axis=1, keepdims=True)
        sum9 = sum9 + _val(m)
    avg = (sum9 - dmin) * 0.125
    acc_ref[2] += jnp.sum(jnp.abs(avg - 0.01) * o_ref[:, 0:1])

    a = jnp.exp(s_ref[...])
    a0 = a[:, 0:1]
    a1 = a[:, 1:2]
    a2 = a[:, 2:3]
    lo = jnp.minimum(jnp.minimum(a0, a1), a2)
    hi = jnp.maximum(jnp.maximum(a0, a1), a2)
    mid = jnp.maximum(jnp.minimum(a0, a1), jnp.minimum(jnp.maximum(a0, a1), a2))
    fr = jnp.log(hi / (lo + 1e-8) + 1e-8)
    disc = 1.0 / (jnp.abs(hi - mid) + 0.001)
    acc_ref[3] += jnp.sum(fr + 0.1 * disc)

    @pl.when(i == nb - 1)
    def _finish():
        out_ref[...] = jnp.array([[acc_ref[0], acc_ref[1], acc_ref[2], acc_ref[3]]],
                                 jnp.float32)


_sc_mesh = plsc.VectorSubcoreMesh(core_axis_name="c", subcore_axis_name="s")


@functools.partial(
    pl.kernel,
    mesh=_sc_mesh,
    out_type=jax.ShapeDtypeStruct((_NW, 2, 16), jnp.float32),
    scratch_types=[
        pltpu.VMEM((7, _N), jnp.float32),
        pltpu.VMEM((16,), jnp.float32),
        pltpu.VMEM((16,), jnp.float32),
    ],
)
def _sc_align(cols_hbm, out_hbm, cols_v, asum_v, cnt_v):
    wid = lax.axis_index("s") * 2 + lax.axis_index("c")
    pltpu.sync_copy(cols_hbm, cols_v)
    base = _N - _SC_ROWS + wid * _RPW

    def group_body(g, group_carry):
        start = base + g * 16
        rvec = [cols_v[k, pl.ds(start, 16)] for k in range(7)]
        acc_g, cnt_g = group_carry
        for r in range(0, 16, 2):
            sca = [rv[r] for rv in rvec]
            scb = [rv[r + 1] for rv in rvec]

            def chunk_body(j, carry):
                a, c = carry
                s = j * 16
                vx = cols_v[0, pl.ds(s, 16)]
                vy = cols_v[1, pl.ds(s, 16)]
                vz = cols_v[2, pl.ds(s, 16)]
                v0 = cols_v[3, pl.ds(s, 16)]
                v1 = cols_v[4, pl.ds(s, 16)]
                v2 = cols_v[5, pl.ds(s, 16)]
                v3 = cols_v[6, pl.ds(s, 16)]
                for sc in (sca, scb):
                    dx = vx - sc[0]
                    dy = vy - sc[1]
                    dz = vz - sc[2]
                    d2 = dx * dx + dy * dy + dz * dz
                    qd = v0 * sc[3] + v1 * sc[4] + v2 * sc[5] + v3 * sc[6]
                    mask = (d2 < 4e-4) & (d2 > 1e-10)
                    a = a + jnp.where(mask, 1.0 - jnp.abs(qd), 0.0)
                    c = c + jnp.where(mask, 1.0, 0.0)
                return a, c

            acc_g, cnt_g = lax.fori_loop(0, _CHUNKS, chunk_body,
                                         (acc_g, cnt_g), unroll=4)
        return acc_g, cnt_g

    zeros = jnp.zeros((16,), jnp.float32)
    acc, cnt = lax.fori_loop(0, _RPW // 16, group_body, (zeros, zeros))
    asum_v[...] = acc
    cnt_v[...] = cnt
    pltpu.sync_copy(asum_v, out_hbm.at[wid, 0])
    pltpu.sync_copy(cnt_v, out_hbm.at[wid, 1])


def kernel(positions, rotations, scales, opacity):
    pT = positions.T
    rT = rotations.T
    cols = jnp.concatenate([pT, rT], axis=0)

    sc_part = _sc_align(cols)

    tc_part = pl.pallas_call(
        _tc_body,
        grid=(_N // _BM,),
        in_specs=[
            pl.BlockSpec((_BM, 3), lambda i: (i, 0)),
            pl.BlockSpec((3, _N), lambda i: (0, 0)),
            pl.BlockSpec((_BM, 4), lambda i: (i, 0)),
            pl.BlockSpec((4, _N), lambda i: (0, 0)),
            pl.BlockSpec((_BM, 3), lambda i: (i, 0)),
            pl.BlockSpec((_BM, 1), lambda i: (i, 0)),
        ],
        out_specs=pl.BlockSpec((1, 4), lambda i: (0, 0)),
        out_shape=jax.ShapeDtypeStruct((1, 4), jnp.float32),
        scratch_shapes=[pltpu.SMEM((4,), jnp.float32)],
    )(positions, pT, rotations, rT, scales, opacity)

    asum = tc_part[0, 0] + jnp.sum(sc_part[:, 0, :])
    cnt = tc_part[0, 1] + jnp.sum(sc_part[:, 1, :])
    flat = -tc_part[0, 3] / _N
    align = jnp.where(cnt > 0.0, asum / jnp.maximum(cnt, 1.0), 0.0)
    dens = tc_part[0, 2] / _N
    return flat + 0.5 * align + 0.2 * dens
